# baseline (speedup 1.0000x reference)
import functools

import jax
import jax.numpy as jnp
from jax import lax
from jax.experimental import pallas as pl
from jax.experimental.pallas import tpu as pltpu
from jax.experimental.pallas import tpu_sc as plsc

_LANES = 16
_GATHER_CHUNK = 128
_UNROLL = 16


def _rsqrt_newton(x):
    i = plsc.bitcast(x, jnp.int32)
    i = jnp.int32(0x5F3759DF) - lax.shift_right_logical(i, 1)
    y = plsc.bitcast(i, jnp.float32)
    half_x = x * jnp.float32(0.5)
    for _ in range(2):
        y = y * (jnp.float32(1.5) - half_x * y * y)
    return y


@functools.lru_cache(maxsize=None)
def _build(batch, dim):
    info = plsc.get_sparse_core_info()
    num_workers = info.num_cores * info.num_subcores
    bpw = batch // num_workers
    nchunks = bpw // _GATHER_CHUNK
    nvec = dim // _LANES
    mesh = plsc.VectorSubcoreMesh(core_axis_name="c", subcore_axis_name="s")

    @functools.partial(
        pl.kernel,
        mesh=mesh,
        compiler_params=pltpu.CompilerParams(needs_layout_passes=False,
                                             use_tc_tiling_on_sc=False),
        out_type=jax.ShapeDtypeStruct((batch,), jnp.float32),
        scratch_types=[
            pltpu.VMEM((bpw,), jnp.int32),
            pltpu.VMEM((bpw,), jnp.int32),
            pltpu.VMEM((bpw,), jnp.int32),
            pltpu.VMEM((bpw, dim), jnp.float32),
            pltpu.VMEM((bpw, dim), jnp.float32),
            pltpu.VMEM((bpw, dim), jnp.float32),
            pltpu.VMEM((bpw,), jnp.float32),
            pltpu.SemaphoreType.DMA,
        ],
    )
    def k(src_hbm, pred_hbm, tail_hbm, ev_hbm, er_hbm, out_hbm,
          si_v, pi_v, ti_v, s_v, r_v, t_v, sc_v, sem):
        wid = lax.axis_index("s") * info.num_cores + lax.axis_index("c")
        base = wid * bpw
        pltpu.sync_copy(src_hbm.at[pl.ds(base, bpw)], si_v)
        pltpu.sync_copy(pred_hbm.at[pl.ds(base, bpw)], pi_v)
        pltpu.sync_copy(tail_hbm.at[pl.ds(base, bpw)], ti_v)
        descs = []
        for c in range(nchunks):
            sl = pl.ds(c * _GATHER_CHUNK, _GATHER_CHUNK)
            descs.append(pltpu.async_copy(ev_hbm.at[si_v.at[sl]], s_v.at[sl], sem))
            descs.append(pltpu.async_copy(er_hbm.at[pi_v.at[sl]], r_v.at[sl], sem))
            descs.append(pltpu.async_copy(ev_hbm.at[ti_v.at[sl]], t_v.at[sl], sem))
        for d in descs:
            d.wait()

        lane_iota = lax.iota(jnp.int32, _LANES)

        def one_row(i):
            s = [s_v[i, pl.ds(v * _LANES, _LANES)] for v in range(nvec)]
            t = [t_v[i, pl.ds(v * _LANES, _LANES)] for v in range(nvec)]
            ssv = s[0] * s[0]
            ttv = t[0] * t[0]
            for v in range(1, nvec):
                ssv = ssv + s[v] * s[v]
                ttv = ttv + t[v] * t[v]
            ss = lax.reduce_sum_p.bind(ssv, axes=(0,))
            tt = lax.reduce_sum_p.bind(ttv, axes=(0,))
            rs = _rsqrt_newton(jnp.broadcast_to(ss, (_LANES,)))
            rt = _rsqrt_newton(jnp.broadcast_to(tt, (_LANES,)))
            r = [r_v[i, pl.ds(v * _LANES, _LANES)] for v in range(nvec)]
            a = jnp.abs(s[0] * rs + r[0] - t[0] * rt)
            for v in range(1, nvec):
                a = a + jnp.abs(s[v] * rs + r[v] - t[v] * rt)
            return -lax.reduce_sum_p.bind(a, axes=(0,))

        def row_block(b, _):
            scores = jnp.zeros((_LANES,), jnp.float32)
            for u in range(_UNROLL):
                val = one_row(b * _UNROLL + u)
                scores = jnp.where(lane_iota == u,
                                   jnp.broadcast_to(val, (_LANES,)), scores)
            sc_v[pl.ds(b * _UNROLL, _UNROLL)] = scores
            return 0

        lax.fori_loop(0, bpw // _UNROLL, row_block, 0)
        pltpu.sync_copy(sc_v, out_hbm.at[pl.ds(base, bpw)])

    return k


def kernel(src, pred, tail, E_v_weight, E_r_weight):
    batch = src.shape[0]
    dim = E_v_weight.shape[1]
    k = _build(batch, dim)
    out = k(src.astype(jnp.int32), pred.astype(jnp.int32),
            tail.astype(jnp.int32), E_v_weight, E_r_weight)
    return out.reshape(batch, 1)
